# baseline (speedup 1.0000x reference)
import functools

import jax
import jax.numpy as jnp
from jax import lax
from jax.experimental import pallas as pl
from jax.experimental.pallas import tpu as pltpu
from jax.experimental.pallas import tpu_sc as plsc

_B = 16384
_D = 32
_NC = 2
_NS = 16
_NW = _NC * _NS
_BPW = _B // _NW
_CHUNK = 128
_NCHUNK = _BPW // _CHUNK

_ROWS_BLK = 2048


def _sc_gather(uidx, midx, user_table, movie_table):
    mesh = plsc.VectorSubcoreMesh(core_axis_name="c", subcore_axis_name="s")

    @functools.partial(
        pl.kernel,
        mesh=mesh,
        compiler_params=pltpu.CompilerParams(use_tc_tiling_on_sc=False),
        out_type=[
            jax.ShapeDtypeStruct((_B, _D), jnp.float32),
            jax.ShapeDtypeStruct((_B, _D), jnp.float32),
        ],
        scratch_types=[
            pltpu.VMEM((_NCHUNK, _CHUNK), jnp.int32),
            pltpu.VMEM((_NCHUNK, _CHUNK), jnp.int32),
            pltpu.VMEM((_BPW, _D), jnp.float32),
            pltpu.VMEM((_BPW, _D), jnp.float32),
            pltpu.SemaphoreType.DMA,
            pltpu.SemaphoreType.DMA,
        ],
    )
    def gather_kernel(uidx_hbm, midx_hbm, utab_hbm, mtab_hbm,
                      uout_hbm, mout_hbm,
                      uidx_v, midx_v, urows_v, mrows_v, usem, msem):
        wid = lax.axis_index("s") * _NC + lax.axis_index("c")
        base = wid * _BPW
        pltpu.sync_copy(uidx_hbm.at[wid], uidx_v)
        pltpu.sync_copy(midx_hbm.at[wid], midx_v)
        ucopies = []
        mcopies = []
        for j in range(_NCHUNK):
            ucopies.append(pltpu.async_copy(
                utab_hbm.at[uidx_v.at[j]],
                urows_v.at[pl.ds(j * _CHUNK, _CHUNK)], usem))
            mcopies.append(pltpu.async_copy(
                mtab_hbm.at[midx_v.at[j]],
                mrows_v.at[pl.ds(j * _CHUNK, _CHUNK)], msem))
        for c in ucopies:
            c.wait()
        pltpu.sync_copy(urows_v, uout_hbm.at[pl.ds(base, _BPW)])
        for c in mcopies:
            c.wait()
        pltpu.sync_copy(mrows_v, mout_hbm.at[pl.ds(base, _BPW)])

    return gather_kernel(uidx, midx, user_table, movie_table)


def _mlp_body(u_ref, m_ref, d_ref, w1e_ref, w1u_ref, w1m_ref, w1d_ref,
              b1_ref, w2_ref, b2_ref, o_ref):
    u = u_ref[...]
    m = m_ref[...]
    d = d_ref[...]
    acc = jnp.dot(u * m, w1e_ref[...], preferred_element_type=jnp.float32)
    acc = acc + jnp.dot(u, w1u_ref[...], preferred_element_type=jnp.float32)
    acc = acc + jnp.dot(m, w1m_ref[...], preferred_element_type=jnp.float32)
    acc = acc + jnp.dot(d, w1d_ref[...], preferred_element_type=jnp.float32)
    h = jnp.maximum(acc + b1_ref[...], 0.0)
    z = jnp.sum(h * w2_ref[...], axis=1, keepdims=True) + b2_ref[...]
    o_ref[...] = 1.0 / (1.0 + jnp.exp(-z))


def kernel(data, user_table, movie_table, W1, b1, W2, b2):
    uidx = data[:, 0].astype(jnp.int32).reshape(_NW, _NCHUNK, _CHUNK)
    midx = data[:, 1].astype(jnp.int32).reshape(_NW, _NCHUNK, _CHUNK)

    u_emb, m_emb = _sc_gather(uidx, midx, user_table, movie_table)

    w1e = W1[0:_D]
    w1u = W1[_D:2 * _D]
    w1m = W1[2 * _D:3 * _D]
    w1d = jnp.concatenate(
        [jnp.zeros((2, W1.shape[1]), W1.dtype), W1[3 * _D:]], axis=0)
    b1r = b1.reshape(1, -1)
    w2r = W2.reshape(1, -1)
    b2r = b2.reshape(1, 1)

    nblk = _B // _ROWS_BLK
    out = pl.pallas_call(
        _mlp_body,
        grid=(nblk,),
        in_specs=[
            pl.BlockSpec((_ROWS_BLK, _D), lambda i: (i, 0)),
            pl.BlockSpec((_ROWS_BLK, _D), lambda i: (i, 0)),
            pl.BlockSpec((_ROWS_BLK, 44), lambda i: (i, 0)),
            pl.BlockSpec((_D, 24), lambda i: (0, 0)),
            pl.BlockSpec((_D, 24), lambda i: (0, 0)),
            pl.BlockSpec((_D, 24), lambda i: (0, 0)),
            pl.BlockSpec((44, 24), lambda i: (0, 0)),
            pl.BlockSpec((1, 24), lambda i: (0, 0)),
            pl.BlockSpec((1, 24), lambda i: (0, 0)),
            pl.BlockSpec((1, 1), lambda i: (0, 0)),
        ],
        out_specs=pl.BlockSpec((_ROWS_BLK, 1), lambda i: (i, 0)),
        out_shape=jax.ShapeDtypeStruct((_B, 1), jnp.float32),
    )(u_emb, m_emb, data, w1e, w1u, w1m, w1d, b1r, w2r, b2r)
    return out


# speedup vs baseline: 10.1536x; 10.1536x over previous
import jax
import jax.numpy as jnp
from jax import lax
from jax.experimental import pallas as pl

_B = 16384
_D = 32
_ROWS_BLK = 2048


def _tdot(lhs_t, w):
    return lax.dot_general(
        lhs_t, w, dimension_numbers=(((0,), (0,)), ((), ())),
        preferred_element_type=jnp.float32)


def _mlp_body(ut_ref, mt_ref, dt_ref,
              w1e_ref, w1u_ref, w1m_ref, w1d_ref,
              b1_ref, w2_ref, b2_ref, o_ref):
    ut = ut_ref[...]
    mt = mt_ref[...]
    dt = dt_ref[...]
    acc = _tdot(ut * mt, w1e_ref[...])
    acc = acc + _tdot(ut, w1u_ref[...])
    acc = acc + _tdot(mt, w1m_ref[...])
    acc = acc + _tdot(dt, w1d_ref[...])
    h = jnp.maximum(acc + b1_ref[...], 0.0)
    z = jnp.sum(h * w2_ref[...], axis=1, keepdims=True) + b2_ref[...]
    o_ref[...] = 1.0 / (1.0 + jnp.exp(-z))


def kernel(data, user_table, movie_table, W1, b1, W2, b2):
    uid = data[:, 0].astype(jnp.int32)
    mid = data[:, 1].astype(jnp.int32)

    u_emb = jnp.take(user_table, uid, axis=0)
    m_emb = jnp.take(movie_table, mid, axis=0)

    w1e = W1[0:_D]
    w1u = W1[_D:2 * _D]
    w1m = W1[2 * _D:3 * _D]
    w1d = jnp.concatenate(
        [jnp.zeros((2, W1.shape[1]), W1.dtype), W1[3 * _D:]], axis=0)
    b1r = b1.reshape(1, -1)
    w2r = W2.reshape(1, -1)
    b2r = b2.reshape(1, 1)

    nblk = _B // _ROWS_BLK
    out = pl.pallas_call(
        _mlp_body,
        grid=(nblk,),
        in_specs=[
            pl.BlockSpec((_D, _ROWS_BLK), lambda i: (0, i)),
            pl.BlockSpec((_D, _ROWS_BLK), lambda i: (0, i)),
            pl.BlockSpec((44, _ROWS_BLK), lambda i: (0, i)),
            pl.BlockSpec((_D, 24), lambda i: (0, 0)),
            pl.BlockSpec((_D, 24), lambda i: (0, 0)),
            pl.BlockSpec((_D, 24), lambda i: (0, 0)),
            pl.BlockSpec((44, 24), lambda i: (0, 0)),
            pl.BlockSpec((1, 24), lambda i: (0, 0)),
            pl.BlockSpec((1, 24), lambda i: (0, 0)),
            pl.BlockSpec((1, 1), lambda i: (0, 0)),
        ],
        out_specs=pl.BlockSpec((_ROWS_BLK, 1), lambda i: (i, 0)),
        out_shape=jax.ShapeDtypeStruct((_B, 1), jnp.float32),
    )(u_emb.T, m_emb.T, data.T, w1e, w1u, w1m, w1d, b1r, w2r, b2r)
    return out


# speedup vs baseline: 12.7588x; 1.2566x over previous
import jax
import jax.numpy as jnp
from jax import lax
from jax.experimental import pallas as pl

_B = 16384
_D = 32
_ROWS_BLK = 2048


def _mlp_body(ut_ref, mt_ref, dt_ref,
              w1e_ref, w1u_ref, w1m_ref, w1d_ref,
              b1_ref, w2_ref, b2_ref, o_ref):
    ut = ut_ref[...]
    mt = mt_ref[...]
    dt = dt_ref[...]
    acc = jnp.dot(w1e_ref[...], ut * mt, preferred_element_type=jnp.float32)
    acc = acc + jnp.dot(w1u_ref[...], ut, preferred_element_type=jnp.float32)
    acc = acc + jnp.dot(w1m_ref[...], mt, preferred_element_type=jnp.float32)
    acc = acc + jnp.dot(w1d_ref[...], dt, preferred_element_type=jnp.float32)
    h = jnp.maximum(acc + b1_ref[...], 0.0)
    z = jnp.sum(h * w2_ref[...], axis=0, keepdims=True) + b2_ref[...]
    o_ref[...] = 1.0 / (1.0 + jnp.exp(-z))


def kernel(data, user_table, movie_table, W1, b1, W2, b2):
    uid = data[:, 0].astype(jnp.int32)
    mid = data[:, 1].astype(jnp.int32)

    u_emb = user_table.at[uid].get(mode="promise_in_bounds")
    m_emb = movie_table.at[mid].get(mode="promise_in_bounds")

    w1e = W1[0:_D].T
    w1u = W1[_D:2 * _D].T
    w1m = W1[2 * _D:3 * _D].T
    w1d = jnp.concatenate(
        [jnp.zeros((2, W1.shape[1]), W1.dtype), W1[3 * _D:]], axis=0).T
    b1c = b1.reshape(-1, 1)
    w2c = W2.reshape(-1, 1)
    b2c = b2.reshape(1, 1)

    nblk = _B // _ROWS_BLK
    out = pl.pallas_call(
        _mlp_body,
        grid=(nblk,),
        in_specs=[
            pl.BlockSpec((_D, _ROWS_BLK), lambda i: (0, i)),
            pl.BlockSpec((_D, _ROWS_BLK), lambda i: (0, i)),
            pl.BlockSpec((44, _ROWS_BLK), lambda i: (0, i)),
            pl.BlockSpec((24, _D), lambda i: (0, 0)),
            pl.BlockSpec((24, _D), lambda i: (0, 0)),
            pl.BlockSpec((24, _D), lambda i: (0, 0)),
            pl.BlockSpec((24, 44), lambda i: (0, 0)),
            pl.BlockSpec((24, 1), lambda i: (0, 0)),
            pl.BlockSpec((24, 1), lambda i: (0, 0)),
            pl.BlockSpec((1, 1), lambda i: (0, 0)),
        ],
        out_specs=pl.BlockSpec((1, _ROWS_BLK), lambda i: (0, i)),
        out_shape=jax.ShapeDtypeStruct((1, _B), jnp.float32),
    )(u_emb.T, m_emb.T, data.T, w1e, w1u, w1m, w1d, b1c, w2c, b2c)
    return out.reshape(_B, 1)


# speedup vs baseline: 13.1998x; 1.0346x over previous
import jax
import jax.numpy as jnp
from jax.experimental import pallas as pl

_B = 16384
_D = 32
_ROWS_BLK = 4096


def _stage1_body(ut_ref, dt_ref, w1u_ref, w1d_ref, b1_ref, p_ref):
    p = jnp.dot(w1u_ref[...], ut_ref[...],
                preferred_element_type=jnp.float32)
    p = p + jnp.dot(w1d_ref[...], dt_ref[...],
                    preferred_element_type=jnp.float32)
    p_ref[...] = p + b1_ref[...]


def _stage2_body(ut_ref, mt_ref, p_ref, w1e_ref, w1m_ref,
                 w2_ref, b2_ref, o_ref):
    ut = ut_ref[...]
    mt = mt_ref[...]
    acc = p_ref[...]
    acc = acc + jnp.dot(w1e_ref[...], ut * mt,
                        preferred_element_type=jnp.float32)
    acc = acc + jnp.dot(w1m_ref[...], mt,
                        preferred_element_type=jnp.float32)
    h = jnp.maximum(acc, 0.0)
    z = jnp.sum(h * w2_ref[...], axis=0, keepdims=True) + b2_ref[...]
    o_ref[...] = 1.0 / (1.0 + jnp.exp(-z))


def kernel(data, user_table, movie_table, W1, b1, W2, b2):
    uid = data[:, 0].astype(jnp.int32)
    mid = data[:, 1].astype(jnp.int32)

    u_emb = user_table.at[uid].get(mode="promise_in_bounds")
    m_emb = movie_table.at[mid].get(mode="promise_in_bounds")

    w1e = W1[0:_D].T
    w1u = W1[_D:2 * _D].T
    w1m = W1[2 * _D:3 * _D].T
    w1d = jnp.concatenate(
        [jnp.zeros((2, W1.shape[1]), W1.dtype), W1[3 * _D:]], axis=0).T
    b1c = b1.reshape(-1, 1)
    w2c = W2.reshape(-1, 1)
    b2c = b2.reshape(1, 1)

    nblk = _B // _ROWS_BLK
    partial = pl.pallas_call(
        _stage1_body,
        grid=(nblk,),
        in_specs=[
            pl.BlockSpec((_D, _ROWS_BLK), lambda i: (0, i)),
            pl.BlockSpec((44, _ROWS_BLK), lambda i: (0, i)),
            pl.BlockSpec((24, _D), lambda i: (0, 0)),
            pl.BlockSpec((24, 44), lambda i: (0, 0)),
            pl.BlockSpec((24, 1), lambda i: (0, 0)),
        ],
        out_specs=pl.BlockSpec((24, _ROWS_BLK), lambda i: (0, i)),
        out_shape=jax.ShapeDtypeStruct((24, _B), jnp.float32),
    )(u_emb.T, data.T, w1u, w1d, b1c)

    out = pl.pallas_call(
        _stage2_body,
        grid=(nblk,),
        in_specs=[
            pl.BlockSpec((_D, _ROWS_BLK), lambda i: (0, i)),
            pl.BlockSpec((_D, _ROWS_BLK), lambda i: (0, i)),
            pl.BlockSpec((24, _ROWS_BLK), lambda i: (0, i)),
            pl.BlockSpec((24, _D), lambda i: (0, 0)),
            pl.BlockSpec((24, _D), lambda i: (0, 0)),
            pl.BlockSpec((24, 1), lambda i: (0, 0)),
            pl.BlockSpec((1, 1), lambda i: (0, 0)),
        ],
        out_specs=pl.BlockSpec((1, _ROWS_BLK), lambda i: (0, i)),
        out_shape=jax.ShapeDtypeStruct((1, _B), jnp.float32),
    )(u_emb.T, m_emb.T, partial, w1e, w1m, w2c, b2c)
    return out.reshape(_B, 1)


# speedup vs baseline: 13.3577x; 1.0120x over previous
import jax
import jax.numpy as jnp
from jax.experimental import pallas as pl

_B = 16384
_D = 32
_ROWS_BLK = 4096
_ROWS_BLK2 = 8192


def _stage1_body(ut_ref, dt_ref, w1u_ref, w1d_ref, b1_ref, p_ref):
    p = jnp.dot(w1u_ref[...], ut_ref[...],
                preferred_element_type=jnp.float32)
    p = p + jnp.dot(w1d_ref[...], dt_ref[...],
                    preferred_element_type=jnp.float32)
    p_ref[...] = p + b1_ref[...]


def _stage2_body(ut_ref, mt_ref, p_ref, w1e_ref, w1m_ref,
                 w2_ref, b2_ref, o_ref):
    ut = ut_ref[...]
    mt = mt_ref[...]
    acc = p_ref[...]
    acc = acc + jnp.dot(w1e_ref[...], ut * mt,
                        preferred_element_type=jnp.float32)
    acc = acc + jnp.dot(w1m_ref[...], mt,
                        preferred_element_type=jnp.float32)
    h = jnp.maximum(acc, 0.0)
    z = jnp.sum(h * w2_ref[...], axis=0, keepdims=True) + b2_ref[...]
    o_ref[...] = 1.0 / (1.0 + jnp.exp(-z))


def kernel(data, user_table, movie_table, W1, b1, W2, b2):
    uid = data[:, 0].astype(jnp.int32)
    mid = data[:, 1].astype(jnp.int32)

    u_emb = user_table.at[uid].get(mode="promise_in_bounds")
    m_emb = movie_table.at[mid].get(mode="promise_in_bounds")

    w1e = W1[0:_D].T
    w1u = W1[_D:2 * _D].T
    w1m = W1[2 * _D:3 * _D].T
    w1d = jnp.concatenate(
        [jnp.zeros((2, W1.shape[1]), W1.dtype), W1[3 * _D:]], axis=0).T
    b1c = b1.reshape(-1, 1)
    w2c = W2.reshape(-1, 1)
    b2c = b2.reshape(1, 1)

    nblk = _B // _ROWS_BLK
    partial = pl.pallas_call(
        _stage1_body,
        grid=(nblk,),
        in_specs=[
            pl.BlockSpec((_D, _ROWS_BLK), lambda i: (0, i)),
            pl.BlockSpec((44, _ROWS_BLK), lambda i: (0, i)),
            pl.BlockSpec((24, _D), lambda i: (0, 0)),
            pl.BlockSpec((24, 44), lambda i: (0, 0)),
            pl.BlockSpec((24, 1), lambda i: (0, 0)),
        ],
        out_specs=pl.BlockSpec((24, _ROWS_BLK), lambda i: (0, i)),
        out_shape=jax.ShapeDtypeStruct((24, _B), jnp.float32),
    )(u_emb.T, data.T, w1u, w1d, b1c)

    out = pl.pallas_call(
        _stage2_body,
        grid=(_B // _ROWS_BLK2,),
        in_specs=[
            pl.BlockSpec((_D, _ROWS_BLK2), lambda i: (0, i)),
            pl.BlockSpec((_D, _ROWS_BLK2), lambda i: (0, i)),
            pl.BlockSpec((24, _ROWS_BLK2), lambda i: (0, i)),
            pl.BlockSpec((24, _D), lambda i: (0, 0)),
            pl.BlockSpec((24, _D), lambda i: (0, 0)),
            pl.BlockSpec((24, 1), lambda i: (0, 0)),
            pl.BlockSpec((1, 1), lambda i: (0, 0)),
        ],
        out_specs=pl.BlockSpec((1, _ROWS_BLK2), lambda i: (0, i)),
        out_shape=jax.ShapeDtypeStruct((1, _B), jnp.float32),
    )(u_emb.T, m_emb.T, partial, w1e, w1m, w2c, b2c)
    return out.reshape(_B, 1)
